# baseline (speedup 1.0000x reference)
import jax
import jax.numpy as jnp
from jax.experimental import pallas as pl
from jax.experimental.pallas import tpu as pltpu

_BLK = 128
_N_CHUNKS = 8


def _copy_fix(fill_ref, x_hbm, o_hbm, diag_vmem, big_sem, in_sem, out_sem):
    n = x_hbm.shape[0]
    nblk = n // _BLK
    chunk = n // _N_CHUNKS
    for c in range(_N_CHUNKS):
        pltpu.make_async_copy(
            x_hbm.at[pl.ds(c * chunk, chunk), :],
            o_hbm.at[pl.ds(c * chunk, chunk), :],
            big_sem,
        ).start()
    for i in range(nblk):
        pltpu.make_async_copy(
            x_hbm.at[pl.ds(i * _BLK, _BLK), pl.ds(i * _BLK, _BLK)],
            diag_vmem.at[i],
            in_sem,
        ).start()
    for i in range(nblk):
        pltpu.make_async_copy(
            x_hbm.at[pl.ds(i * _BLK, _BLK), pl.ds(i * _BLK, _BLK)],
            diag_vmem.at[i],
            in_sem,
        ).wait()
    rows = jax.lax.broadcasted_iota(jnp.int32, (_BLK, _BLK), 0)
    cols = jax.lax.broadcasted_iota(jnp.int32, (_BLK, _BLK), 1)
    mask = (rows == cols)[None, :, :]
    diag_vmem[...] = jnp.where(mask, fill_ref[0, 0], diag_vmem[...])
    for c in range(_N_CHUNKS):
        pltpu.make_async_copy(
            x_hbm.at[pl.ds(c * chunk, chunk), :],
            o_hbm.at[pl.ds(c * chunk, chunk), :],
            big_sem,
        ).wait()
    for i in range(nblk):
        pltpu.make_async_copy(
            diag_vmem.at[i],
            o_hbm.at[pl.ds(i * _BLK, _BLK), pl.ds(i * _BLK, _BLK)],
            out_sem,
        ).start()
    for i in range(nblk):
        pltpu.make_async_copy(
            diag_vmem.at[i],
            o_hbm.at[pl.ds(i * _BLK, _BLK), pl.ds(i * _BLK, _BLK)],
            out_sem,
        ).wait()


def kernel(x, fill_value):
    n = min(x.shape)
    fill = jnp.asarray(fill_value, x.dtype).reshape(1, 1)
    return pl.pallas_call(
        _copy_fix,
        in_specs=[
            pl.BlockSpec(memory_space=pltpu.SMEM),
            pl.BlockSpec(memory_space=pl.MemorySpace.ANY),
        ],
        out_specs=pl.BlockSpec(memory_space=pl.MemorySpace.ANY),
        out_shape=jax.ShapeDtypeStruct(x.shape, x.dtype),
        scratch_shapes=[
            pltpu.VMEM((n // _BLK, _BLK, _BLK), x.dtype),
            pltpu.SemaphoreType.DMA,
            pltpu.SemaphoreType.DMA,
            pltpu.SemaphoreType.DMA,
        ],
    )(fill, x)


# speedup vs baseline: 48.9076x; 48.9076x over previous
import jax
import jax.numpy as jnp
from jax.experimental import pallas as pl
from jax.experimental.pallas import tpu as pltpu

_BLOCK_ROWS = 256


def _fill_diag_block(fill_ref, x_ref, o_ref):
    i = pl.program_id(0)
    o_ref[...] = x_ref[...]
    cols = pl.ds(i * _BLOCK_ROWS, _BLOCK_ROWS)
    sub = x_ref[:, cols]
    r = jax.lax.broadcasted_iota(jnp.int32, (_BLOCK_ROWS, _BLOCK_ROWS), 0)
    c = jax.lax.broadcasted_iota(jnp.int32, (_BLOCK_ROWS, _BLOCK_ROWS), 1)
    o_ref[:, cols] = jnp.where(r == c, fill_ref[0], sub)


def kernel(x, fill_value):
    n_rows, n_cols = x.shape
    fill = jnp.asarray(fill_value, x.dtype).reshape(1)
    return pl.pallas_call(
        _fill_diag_block,
        grid=(n_rows // _BLOCK_ROWS,),
        in_specs=[
            pl.BlockSpec(memory_space=pltpu.SMEM),
            pl.BlockSpec((_BLOCK_ROWS, n_cols), lambda i: (i, 0)),
        ],
        out_specs=pl.BlockSpec((_BLOCK_ROWS, n_cols), lambda i: (i, 0)),
        out_shape=jax.ShapeDtypeStruct(x.shape, x.dtype),
    )(fill, x)
